# Initial kernel scaffold; baseline (speedup 1.0000x reference)
import jax
import jax.numpy as jnp
from jax.experimental import pallas as pl


def kernel(h, edge_index, edge_attr, batch, params):
    raise NotImplementedError("write your pallas kernel here")


# speedup vs baseline: 1.8661x; 1.8661x over previous
import functools

import jax
import jax.numpy as jnp
from jax import lax
from jax.experimental import pallas as pl
from jax.experimental.pallas import tpu as pltpu
from jax.experimental.pallas import tpu_sc as plsc

HALF = 128
R = 400
EB = 4000


def _sc_partition(src, dst, n):
    e = dst.shape[0]
    rows_pt = 624
    tid = jnp.clip(dst // rows_pt, 0, 15).astype(jnp.int32)
    perm = jnp.argsort(tid, stable=True).astype(jnp.int32)
    st = tid[perm]
    off16 = jnp.searchsorted(st, jnp.arange(16, dtype=jnp.int32),
                             side="left").astype(jnp.int32)
    off17 = jnp.concatenate([off16, jnp.full((1,), e, jnp.int32)])
    pairs = jnp.zeros((16, 8), jnp.int32)
    pairs = pairs.at[:, 0].set(off17[:16]).at[:, 1].set(off17[1:])
    off = jnp.concatenate([pairs.reshape(-1),
                           jnp.zeros((16,), jnp.int32)])
    pad = jnp.zeros((192,), jnp.int32)
    src_s = jnp.concatenate([src[perm], pad])
    dst_s = jnp.concatenate([dst[perm], pad])
    eid_s = jnp.concatenate([perm, pad])
    return src_s, dst_s, eid_s, off


def _sc_scatter(x_split, ea_split, src_s, dst_s, eid_s, off):
    n = x_split.shape[1]
    ns = 16
    rows_pt = 624
    extra = n - ns * rows_pt
    zb = 48
    nz = rows_pt // zb
    G = 128
    trash = n

    mesh = plsc.VectorSubcoreMesh(core_axis_name="c", subcore_axis_name="s",
                                  num_cores=2, num_subcores=ns)

    @functools.partial(
        pl.kernel,
        out_type=jax.ShapeDtypeStruct((2, n, HALF), jnp.float32),
        mesh=mesh,
        scratch_types=[
            pltpu.VMEM((16,), jnp.int32),
            pltpu.VMEM((G,), jnp.int32),
            pltpu.VMEM((G,), jnp.int32),
            pltpu.VMEM((G,), jnp.int32),
            pltpu.VMEM((G, HALF), jnp.float32),
            pltpu.VMEM((G, HALF), jnp.float32),
            pltpu.VMEM((zb, HALF), jnp.float32),
            pltpu.VMEM_SHARED((n + 16, HALF), jnp.float32),
            pltpu.SemaphoreType.DMA,
        ],
    )
    def body(x_hbm, ea_hbm, src_hbm, dst_hbm, eid_hbm, off_hbm, out_hbm,
             offv, fsrc, fdst, feid, xb, ebuf, zerob, agg, sem):
        c = lax.axis_index("c")
        s = lax.axis_index("s")
        zeros16 = jnp.zeros((16,), jnp.float32)
        iota16 = lax.iota(jnp.int32, 16)

        def zrow(i, _):
            for k in range(HALF // 16):
                zerob[i, pl.ds(k * 16, 16)] = zeros16
            return 0
        lax.fori_loop(0, zb, zrow, 0)
        for k in range(nz):
            pltpu.sync_copy(zerob, agg.at[pl.ds(s * rows_pt + k * zb, zb)])

        @pl.when(s == ns - 1)
        def _():
            pltpu.sync_copy(zerob.at[pl.ds(0, extra)],
                            agg.at[pl.ds(ns * rows_pt, extra)])

        pltpu.sync_copy(off_hbm.at[pl.ds(s * 8, 16)], offv)
        vec = offv[...]
        beg = vec[0]
        end = vec[1]
        b0 = (beg // 8) * 8
        nch = jnp.maximum(end - b0 + (G - 1), 0) // G

        def chunk(j, _):
            b = pl.multiple_of(b0 + j * G, 8)
            pltpu.sync_copy(src_hbm.at[pl.ds(b, G)], fsrc)
            pltpu.sync_copy(eid_hbm.at[pl.ds(b, G)], feid)
            pltpu.sync_copy(dst_hbm.at[pl.ds(b, G)], fdst)
            for k in range(G // 16):
                sl = pl.ds(k * 16, 16)
                pos = b + k * 16 + iota16
                inside = (pos >= beg) & (pos < end)
                fdst[sl] = jnp.where(inside, fdst[sl], trash)

            pltpu.async_copy(x_hbm.at[c].at[fsrc], xb, sem).wait()
            pltpu.async_copy(ea_hbm.at[c].at[feid], ebuf, sem).wait()

            def rowfn(i, _):
                for k in range(HALF // 16):
                    sl = (i, pl.ds(k * 16, 16))
                    xb[sl] = jnp.maximum(xb[sl] + ebuf[sl], 0.0)
                return 0
            lax.fori_loop(0, G, rowfn, 0)
            pltpu.sync_copy(xb, agg.at[fdst], add=True)
            return 0
        lax.fori_loop(0, nch, chunk, 0)

        pltpu.sync_copy(agg.at[pl.ds(s * rows_pt, rows_pt)],
                        out_hbm.at[c, pl.ds(s * rows_pt, rows_pt)])

        @pl.when(s == ns - 1)
        def _():
            pltpu.sync_copy(agg.at[pl.ds(ns * rows_pt, extra)],
                            out_hbm.at[c, pl.ds(ns * rows_pt, extra)])

    return body(x_split, ea_split, src_s, dst_s, eid_s, off)


def _full(shape):
    return pl.BlockSpec(shape, lambda *a: (0,) * len(shape))


def _ea_kernel(edge_attr, w, b):
    e = edge_attr.shape[0]

    def body(a_ref, w_ref, b_ref, out_ref):
        z = jnp.dot(a_ref[...], w_ref[...],
                    preferred_element_type=jnp.float32,
                    precision=lax.Precision.HIGHEST) + b_ref[...]
        out_ref[0] = z[:, :HALF]
        out_ref[1] = z[:, HALF:]

    return pl.pallas_call(
        body,
        grid=(e // EB,),
        in_specs=[
            pl.BlockSpec((EB, edge_attr.shape[1]), lambda i: (i, 0)),
            _full(w.shape),
            _full((1, w.shape[1])),
        ],
        out_specs=pl.BlockSpec((2, EB, HALF), lambda i: (0, i, 0)),
        out_shape=jax.ShapeDtypeStruct((2, e, HALF), jnp.float32),
    )(edge_attr, w, b.reshape(1, -1))


def _split_kernel(x):
    n = x.shape[0]

    def body(x_ref, out_ref):
        out_ref[0] = x_ref[:, :HALF]
        out_ref[1] = x_ref[:, HALF:]

    return pl.pallas_call(
        body,
        grid=(n // R,),
        in_specs=[pl.BlockSpec((R, 2 * HALF), lambda i: (i, 0))],
        out_specs=pl.BlockSpec((2, R, HALF), lambda i: (0, i, 0)),
        out_shape=jax.ShapeDtypeStruct((2, n, HALF), jnp.float32),
    )(x)


def _k1(x, agg, w, b):
    n = x.shape[0]
    h = w.shape[1]

    def body(x_ref, a_ref, w_ref, b_ref, z_ref, st_ref):
        z = x_ref[...] + jnp.concatenate([a_ref[0], a_ref[1]], axis=1)
        z = jnp.dot(z, w_ref[...], preferred_element_type=jnp.float32)
        z = z + b_ref[...]
        z_ref[...] = z

        @pl.when(pl.program_id(0) == 0)
        def _():
            st_ref[...] = jnp.zeros_like(st_ref)
        st_ref[0:1, :] += jnp.sum(z, axis=0, keepdims=True)
        st_ref[1:2, :] += jnp.sum(z * z, axis=0, keepdims=True)

    return pl.pallas_call(
        body,
        grid=(n // R,),
        in_specs=[
            pl.BlockSpec((R, x.shape[1]), lambda i: (i, 0)),
            pl.BlockSpec((2, R, HALF), lambda i: (0, i, 0)),
            _full(w.shape),
            _full((1, h)),
        ],
        out_specs=[
            pl.BlockSpec((R, h), lambda i: (i, 0)),
            _full((2, h)),
        ],
        out_shape=[
            jax.ShapeDtypeStruct((n, h), jnp.float32),
            jax.ShapeDtypeStruct((2, h), jnp.float32),
        ],
    )(x, agg, w, b.reshape(1, -1))


def _k_norm_mm(z, st, g, be, w, b):
    n = z.shape[0]
    h = w.shape[1]

    def body(z_ref, st_ref, g_ref, be_ref, w_ref, b_ref, o_ref, so_ref):
        m = st_ref[0:1, :] / n
        v = st_ref[1:2, :] / n - m * m
        a = jnp.maximum(
            (z_ref[...] - m) / jnp.sqrt(v + 1e-5) * g_ref[...] + be_ref[...],
            0.0)
        o = jnp.dot(a, w_ref[...], preferred_element_type=jnp.float32)
        o = o + b_ref[...]
        o_ref[...] = o

        @pl.when(pl.program_id(0) == 0)
        def _():
            so_ref[...] = jnp.zeros_like(so_ref)
        so_ref[0:1, :] += jnp.sum(o, axis=0, keepdims=True)
        so_ref[1:2, :] += jnp.sum(o * o, axis=0, keepdims=True)

    return pl.pallas_call(
        body,
        grid=(n // R,),
        in_specs=[
            pl.BlockSpec((R, z.shape[1]), lambda i: (i, 0)),
            _full((2, z.shape[1])),
            _full((1, z.shape[1])),
            _full((1, z.shape[1])),
            _full(w.shape),
            _full((1, h)),
        ],
        out_specs=[
            pl.BlockSpec((R, h), lambda i: (i, 0)),
            _full((2, h)),
        ],
        out_shape=[
            jax.ShapeDtypeStruct((n, h), jnp.float32),
            jax.ShapeDtypeStruct((2, h), jnp.float32),
        ],
    )(z, st, g.reshape(1, -1), be.reshape(1, -1), w, b.reshape(1, -1))


def _k_update(z, st, g, be, res):
    n = z.shape[0]
    h = z.shape[1]
    has_res = res is not None

    def body(*refs):
        if has_res:
            z_ref, st_ref, g_ref, be_ref, r_ref, x_ref, xs_ref = refs
        else:
            z_ref, st_ref, g_ref, be_ref, x_ref, xs_ref = refs
        m = st_ref[0:1, :] / n
        v = st_ref[1:2, :] / n - m * m
        a = (z_ref[...] - m) / jnp.sqrt(v + 1e-5) * g_ref[...] + be_ref[...]
        if has_res:
            a = a + r_ref[...]
        x = jnp.maximum(a, 0.0)
        x_ref[...] = x
        xs_ref[0] = x[:, :HALF]
        xs_ref[1] = x[:, HALF:]

    in_specs = [
        pl.BlockSpec((R, h), lambda i: (i, 0)),
        _full((2, h)),
        _full((1, h)),
        _full((1, h)),
    ]
    args = [z, st, g.reshape(1, -1), be.reshape(1, -1)]
    if has_res:
        in_specs.append(pl.BlockSpec((R, h), lambda i: (i, 0)))
        args.append(res)

    return pl.pallas_call(
        body,
        grid=(n // R,),
        in_specs=in_specs,
        out_specs=[
            pl.BlockSpec((R, h), lambda i: (i, 0)),
            pl.BlockSpec((2, R, HALF), lambda i: (0, i, 0)),
        ],
        out_shape=[
            jax.ShapeDtypeStruct((n, h), jnp.float32),
            jax.ShapeDtypeStruct((2, n, HALF), jnp.float32),
        ],
    )(*args)


def _k_pool(x, batch2d):
    n, h = x.shape
    ng = 64

    def body(x_ref, b_ref, s_ref, c_ref):
        oh = jnp.where(
            b_ref[...] == lax.broadcasted_iota(jnp.int32, (1, ng), 1),
            1.0, 0.0)
        @pl.when(pl.program_id(0) == 0)
        def _():
            s_ref[...] = jnp.zeros_like(s_ref)
            c_ref[...] = jnp.zeros_like(c_ref)
        dn = (((0,), (0,)), ((), ()))
        s_ref[...] += lax.dot_general(oh, x_ref[...], dn,
                                      preferred_element_type=jnp.float32,
                                      precision=lax.Precision.HIGHEST)
        c_ref[...] += lax.dot_general(oh, jnp.ones_like(x_ref[...]), dn,
                                      preferred_element_type=jnp.float32,
                                      precision=lax.Precision.HIGHEST)

    return pl.pallas_call(
        body,
        grid=(n // R,),
        in_specs=[
            pl.BlockSpec((R, h), lambda i: (i, 0)),
            pl.BlockSpec((R, 1), lambda i: (i, 0)),
        ],
        out_specs=[_full((ng, h)), _full((ng, h))],
        out_shape=[
            jax.ShapeDtypeStruct((ng, h), jnp.float32),
            jax.ShapeDtypeStruct((ng, h), jnp.float32),
        ],
    )(x, batch2d)


def _k_final(sums, cnts, p):
    ng = sums.shape[0]

    def bn(t, g, b):
        m = jnp.mean(t, axis=0, keepdims=True)
        v = jnp.mean(t * t, axis=0, keepdims=True) - m * m
        return (t - m) / jnp.sqrt(v + 1e-5) * g + b

    def body(s_ref, c_ref, w1, b1, g1, be1, w2, b2, g2, be2, w3, b3, o_ref):
        gf = s_ref[...] / jnp.maximum(c_ref[...], 1.0)
        t = jnp.dot(gf, w1[...], preferred_element_type=jnp.float32) + b1[...]
        t = jnp.maximum(bn(t, g1[...], be1[...]), 0.0)
        t = jnp.dot(t, w2[...], preferred_element_type=jnp.float32) + b2[...]
        t = jnp.maximum(bn(t, g2[...], be2[...]), 0.0)
        o_ref[...] = jnp.dot(t, w3[...],
                             preferred_element_type=jnp.float32) + b3[...]

    hid = p["W1"].shape[1]
    out_dim = p["W3"].shape[1]
    return pl.pallas_call(
        body,
        in_specs=[
            _full(sums.shape), _full(cnts.shape),
            _full(p["W1"].shape), _full((1, hid)),
            _full((1, hid)), _full((1, hid)),
            _full(p["W2"].shape), _full((1, hid)),
            _full((1, hid)), _full((1, hid)),
            _full(p["W3"].shape), _full((1, out_dim)),
        ],
        out_specs=_full((ng, out_dim)),
        out_shape=jax.ShapeDtypeStruct((ng, out_dim), jnp.float32),
    )(sums, cnts,
      p["W1"], p["b1"].reshape(1, -1), p["g1"].reshape(1, -1),
      p["be1"].reshape(1, -1),
      p["W2"], p["b2"].reshape(1, -1), p["g2"].reshape(1, -1),
      p["be2"].reshape(1, -1),
      p["W3"], p["b3"].reshape(1, -1))


def kernel(h, edge_index, edge_attr, batch, params):
    src = edge_index[0].astype(jnp.int32)
    dst = edge_index[1].astype(jnp.int32)
    batch2d = batch.astype(jnp.int32).reshape(-1, 1)

    n = h.shape[0]
    src_s, dst_s, eid_s, off = _sc_partition(src, dst, n)
    ea_split = _ea_kernel(edge_attr, params["edge_W"], params["edge_b"])
    x = h
    x_split = _split_kernel(h)
    res = None
    for i in range(5):
        p = params["convs"][i]
        agg = _sc_scatter(x_split, ea_split, src_s, dst_s, eid_s, off)
        z1, s1 = _k1(x, agg, p["W1"], p["b1"])
        z2, s2 = _k_norm_mm(z1, s1, p["g1"], p["be1"], p["W2"], p["b2"])
        z3, s3 = _k_norm_mm(z2, s2, p["g2"], p["be2"], p["W3"], p["b3"])
        g, b = params["norms"][i]
        x, x_split = _k_update(z3, s3, g, b, res)
        res = x

    sums, cnts = _k_pool(x, batch2d)
    return _k_final(sums, cnts, params["final"])
